# baseline (speedup 1.0000x reference)
import functools
import math

import jax
import jax.numpy as jnp
from jax import lax
from jax.experimental import pallas as pl
from jax.experimental.pallas import tpu as pltpu
from jax.experimental.pallas import tpu_sc as plsc

N = 10000
E = 320000
C = 128
H = 8
D = C // H
HB = H // 2
CH = C // 2

NS = 16
EW = E // NS
CHUNK = 32
NCHUNK = EW // CHUNK
NG = CHUNK // 16
SUPER = 25
NSUP = NCHUNK // SUPER
AGG_ROWS = 10240
ROWS_PER_TILE = AGG_ROWS // NS
ZCH = 64
BLK = 1000


def _prologue_body(x_ref, wqkv_ref, bqkv_ref, g_ref, b_ref, q_ref, kv_ref):
    x = x_ref[...]
    mu = jnp.mean(x, axis=-1, keepdims=True)
    var = jnp.mean((x - mu) * (x - mu), axis=-1, keepdims=True)
    xn = (x - mu) * lax.rsqrt(var + 1e-5) * g_ref[...] + b_ref[...]
    qkv = jnp.dot(xn, wqkv_ref[...], preferred_element_type=jnp.float32)
    qkv = qkv + bqkv_ref[...]
    q_ref[...] = qkv[:, 0 * C:1 * C]
    kv_ref[...] = jnp.stack((
        jnp.concatenate((qkv[:, 128:192], qkv[:, 256:320]), axis=1),
        jnp.concatenate((qkv[:, 192:256], qkv[:, 320:384]), axis=1),
    ))


def _prologue(x, wqkv, bqkv, g, b):
    return pl.pallas_call(
        _prologue_body,
        grid=(N // BLK,),
        in_specs=[
            pl.BlockSpec((BLK, C), lambda i: (i, 0)),
            pl.BlockSpec((C, 3 * C), lambda i: (0, 0)),
            pl.BlockSpec((1, 3 * C), lambda i: (0, 0)),
            pl.BlockSpec((1, C), lambda i: (0, 0)),
            pl.BlockSpec((1, C), lambda i: (0, 0)),
        ],
        out_specs=[pl.BlockSpec((BLK, C), lambda i: (i, 0)),
                   pl.BlockSpec((2, BLK, C), lambda i: (0, i, 0))],
        out_shape=[jax.ShapeDtypeStruct((N, C), jnp.float32),
                   jax.ShapeDtypeStruct((2, N, C), jnp.float32)],
    )(x, wqkv, bqkv, g, b)


def _sc_body(q_hbm, kv_hbm, src_hbm, dst_hbm, agg_out, z_out,
             src_sv, dst_sv, row_a, row_b, qa, kva, wa, qb, kvb, wb,
             zacc, zbuf, aggsh, sq_a, skv_a, sq_b, skv_b, sw_a, sw_b):
    c = lax.axis_index("c")
    s = lax.axis_index("s")
    cb = c * CH

    zero16 = jnp.zeros((16,), jnp.float32)
    izero16 = jnp.zeros((16,), jnp.int32)

    def _zrow(r, _):
        for j in range(CH // 16):
            zbuf[r, pl.ds(j * 16, 16)] = zero16
        return 0
    lax.fori_loop(0, ZCH, _zrow, 0)
    for i in range(ROWS_PER_TILE // ZCH):
        pltpu.sync_copy(zbuf, aggsh.at[pl.ds(s * ROWS_PER_TILE + i * ZCH, ZCH)])
    for j in range(C // 16):
        zacc[pl.ds(j * 16, 16)] = zero16
    for g in range(NG):
        gs = pl.ds(g * 16, 16)
        row_a[gs] = izero16
        row_b[gs] = izero16
    for e in range(CHUNK):
        for j in range(CH // 16):
            wa[e, pl.ds(j * 16, 16)] = zero16
            wb[e, pl.ds(j * 16, 16)] = zero16
    plsc.subcore_barrier()
    pltpu.async_copy(wa, aggsh.at[row_a], sw_a, add=True)
    pltpu.async_copy(wb, aggsh.at[row_b], sw_b, add=True)

    lanes = lax.iota(jnp.int32, 16)

    kvh = kv_hbm.at[c]

    def issue(cc, qd, kvd, semq, semkv):
        o = cc * CHUNK
        di = dst_sv.at[pl.ds(o, CHUNK)]
        si = src_sv.at[pl.ds(o, CHUNK)]
        pltpu.async_copy(q_hbm.at[di], qd, semq)
        pltpu.async_copy(kvh.at[si], kvd, semkv)

    def wait(qd, kvd, semq, semkv):
        dummy = q_hbm.at[pl.ds(0, CHUNK)]
        pltpu.make_async_copy(dummy, qd, semq).wait()
        pltpu.make_async_copy(dummy, kvd, semkv).wait()

    def compute(cc, qd, kvd, wd, rowd, semw):
        pltpu.make_async_copy(wd, aggsh.at[rowd], semw).wait()
        for g in range(NG):
            evec = g * 16 + lanes
            gs = pl.ds(cc * CHUNK + g * 16, 16)
            rowd[pl.ds(g * 16, 16)] = dst_sv[gs]
            for h in range(HB):
                acc = zero16
                for d in range(D):
                    coll = jnp.full((16,), h * D + d, jnp.int32)
                    qv = plsc.load_gather(qd, [evec, coll + cb])
                    kv = plsc.load_gather(kvd, [evec, coll])
                    acc = acc + qv * kv
                p = jnp.exp(acc * 0.25)
                hs = pl.ds((c * HB + h) * 16, 16)
                zacc[hs] = zacc[hs] + p
                for d in range(D):
                    coll = jnp.full((16,), h * D + d, jnp.int32)
                    wv = plsc.load_gather(kvd, [evec, coll + CH]) * p
                    plsc.store_scatter(wd, [evec, coll], wv)
        pltpu.async_copy(wd, aggsh.at[rowd], semw, add=True)

    def super_body(t, _):
        base = s * EW + t * (SUPER * CHUNK)
        pltpu.sync_copy(src_hbm.at[pl.ds(base, SUPER * CHUNK)], src_sv)
        pltpu.sync_copy(dst_hbm.at[pl.ds(base, SUPER * CHUNK)], dst_sv)

        issue(0, qa, kva, sq_a, skv_a)

        def pair_body(i, _):
            issue(2 * i + 1, qb, kvb, sq_b, skv_b)
            wait(qa, kva, sq_a, skv_a)
            compute(2 * i, qa, kva, wa, row_a, sw_a)
            issue(2 * i + 2, qa, kva, sq_a, skv_a)
            wait(qb, kvb, sq_b, skv_b)
            compute(2 * i + 1, qb, kvb, wb, row_b, sw_b)
            return 0
        lax.fori_loop(0, SUPER // 2, pair_body, 0)

        wait(qa, kva, sq_a, skv_a)
        compute(SUPER - 1, qa, kva, wa, row_a, sw_a)
        return 0

    lax.fori_loop(0, NSUP, super_body, 0)

    pltpu.make_async_copy(wa, aggsh.at[row_a], sw_a).wait()
    pltpu.make_async_copy(wb, aggsh.at[row_b], sw_b).wait()

    pltpu.sync_copy(zacc, z_out.at[c * NS + s])
    plsc.subcore_barrier()
    pltpu.sync_copy(aggsh.at[pl.ds(s * ROWS_PER_TILE, ROWS_PER_TILE)],
                    agg_out.at[c, pl.ds(s * ROWS_PER_TILE, ROWS_PER_TILE)])


@functools.partial(
    pl.kernel,
    out_type=(jax.ShapeDtypeStruct((2, AGG_ROWS, CH), jnp.float32),
              jax.ShapeDtypeStruct((2 * NS, C), jnp.float32)),
    mesh=plsc.VectorSubcoreMesh(core_axis_name="c", subcore_axis_name="s"),
    scratch_types=(
        [pltpu.VMEM((SUPER * CHUNK,), jnp.int32)] * 2
        + [pltpu.VMEM((CHUNK,), jnp.int32)] * 2
        + [pltpu.VMEM((CHUNK, C), jnp.float32)] * 2
        + [pltpu.VMEM((CHUNK, CH), jnp.float32)]
        + [pltpu.VMEM((CHUNK, C), jnp.float32)] * 2
        + [pltpu.VMEM((CHUNK, CH), jnp.float32)]
        + [pltpu.VMEM((C,), jnp.float32),
           pltpu.VMEM((ZCH, CH), jnp.float32),
           pltpu.VMEM_SHARED((AGG_ROWS, CH), jnp.float32)]
        + [pltpu.SemaphoreType.DMA] * 6
    ),
    compiler_params=pltpu.CompilerParams(needs_layout_passes=False),
)
def _sc_attention(q_hbm, kv_hbm, src_hbm, dst_hbm, agg_out, z_out, *rest):
    _sc_body(q_hbm, kv_hbm, src_hbm, dst_hbm, agg_out, z_out, *rest)


def _epilogue_body(x_ref, a0_ref, a1_ref, scale_ref, wo_ref, bo_ref,
                   g2_ref, b2_ref, w1_ref, b1_ref, w2_ref, bf2_ref, out_ref):
    agg = jnp.concatenate((a0_ref[...], a1_ref[...]), axis=1) * scale_ref[...]
    o = jnp.dot(agg, wo_ref[...], preferred_element_type=jnp.float32)
    x2 = x_ref[...] + o + bo_ref[...]
    mu = jnp.mean(x2, axis=-1, keepdims=True)
    var = jnp.mean((x2 - mu) * (x2 - mu), axis=-1, keepdims=True)
    xn2 = (x2 - mu) * lax.rsqrt(var + 1e-5) * g2_ref[...] + b2_ref[...]
    h = jnp.dot(xn2, w1_ref[...], preferred_element_type=jnp.float32)
    h = jnp.maximum(h + b1_ref[...], 0.0)
    f = jnp.dot(h, w2_ref[...], preferred_element_type=jnp.float32)
    out_ref[...] = x2 + f + bf2_ref[...]


def _epilogue(x, a0, a1, scale, wo, bo, g2, b2, w1, b1, w2, bf2):
    full = lambda r, c: pl.BlockSpec((r, c), lambda i: (0, 0))
    return pl.pallas_call(
        _epilogue_body,
        grid=(N // BLK,),
        in_specs=[
            pl.BlockSpec((BLK, C), lambda i: (i, 0)),
            pl.BlockSpec((BLK, CH), lambda i: (i, 0)),
            pl.BlockSpec((BLK, CH), lambda i: (i, 0)),
            full(1, C),
            full(C, C),
            full(1, C),
            full(1, C),
            full(1, C),
            full(C, 4 * C),
            full(1, 4 * C),
            full(4 * C, C),
            full(1, C),
        ],
        out_specs=pl.BlockSpec((BLK, C), lambda i: (i, 0)),
        out_shape=jax.ShapeDtypeStruct((N, C), jnp.float32),
    )(x, a0, a1, scale, wo, bo, g2, b2, w1, b1, w2, bf2)


def kernel(x, edge_index, Wq, bq, Wk, bk, Wv, bv, Wo, bo, ln1_g, ln1_b,
           ln2_g, ln2_b, W1, b1, W2, b2):
    wqkv = jnp.concatenate([Wq, Wk, Wv], axis=1)
    bqkv = jnp.concatenate([bq, bk, bv]).reshape(1, 3 * C)
    q, kvp = _prologue(x, wqkv, bqkv, ln1_g.reshape(1, C), ln1_b.reshape(1, C))

    src = edge_index[0].astype(jnp.int32)
    dst = edge_index[1].astype(jnp.int32)
    aggp, zp = _sc_attention(q, kvp, src, dst)

    z = zp.sum(axis=0).reshape(H, D).sum(-1)
    scale = jnp.repeat(1.0 / z, D).reshape(1, C)

    return _epilogue(x, aggp[0], aggp[1], scale, Wo, bo.reshape(1, C),
                     ln2_g.reshape(1, C), ln2_b.reshape(1, C),
                     W1, b1.reshape(1, 4 * C), W2, b2.reshape(1, C))


# speedup vs baseline: 1.1903x; 1.1903x over previous
import functools
import math

import jax
import jax.numpy as jnp
from jax import lax
from jax.experimental import pallas as pl
from jax.experimental.pallas import tpu as pltpu
from jax.experimental.pallas import tpu_sc as plsc

N = 10000
E = 320000
C = 128
H = 8
D = C // H
HB = H // 2
CH = C // 2

NS = 16
EW = E // NS
CHUNK = 32
NCHUNK = EW // CHUNK
NG = CHUNK // 16
SUPER = 25
NSUP = NCHUNK // SUPER
AGG_ROWS = 10240
ROWS_PER_TILE = AGG_ROWS // NS
ZCH = 64
BLK = 1000


def _prologue_body(x_ref, wqkv_ref, bqkv_ref, g_ref, b_ref, q_ref, kv_ref):
    x = x_ref[...]
    mu = jnp.mean(x, axis=-1, keepdims=True)
    var = jnp.mean((x - mu) * (x - mu), axis=-1, keepdims=True)
    xn = (x - mu) * lax.rsqrt(var + 1e-5) * g_ref[...] + b_ref[...]
    qkv = jnp.dot(xn, wqkv_ref[...], preferred_element_type=jnp.float32)
    qkv = qkv + bqkv_ref[...]
    q_ref[...] = qkv[:, 0 * C:1 * C]
    kv_ref[...] = jnp.stack((
        jnp.concatenate((qkv[:, 128:192], qkv[:, 256:320]), axis=1),
        jnp.concatenate((qkv[:, 192:256], qkv[:, 320:384]), axis=1),
    ))


def _prologue(x, wqkv, bqkv, g, b):
    return pl.pallas_call(
        _prologue_body,
        grid=(N // BLK,),
        in_specs=[
            pl.BlockSpec((BLK, C), lambda i: (i, 0)),
            pl.BlockSpec((C, 3 * C), lambda i: (0, 0)),
            pl.BlockSpec((1, 3 * C), lambda i: (0, 0)),
            pl.BlockSpec((1, C), lambda i: (0, 0)),
            pl.BlockSpec((1, C), lambda i: (0, 0)),
        ],
        out_specs=[pl.BlockSpec((BLK, C), lambda i: (i, 0)),
                   pl.BlockSpec((2, BLK, C), lambda i: (0, i, 0))],
        out_shape=[jax.ShapeDtypeStruct((N, C), jnp.float32),
                   jax.ShapeDtypeStruct((2, N, C), jnp.float32)],
    )(x, wqkv, bqkv, g, b)


def _sc_body(q_hbm, kv_hbm, src_hbm, dst_hbm, agg_out, z_out,
             src_sv, dst_sv, row_a, row_b, qa, kva, wa, qb, kvb, wb,
             zacc, zbuf, aggsh, sq_a, skv_a, sq_b, skv_b, sw_a, sw_b):
    c = lax.axis_index("c")
    s = lax.axis_index("s")
    cb = c * CH

    zero16 = jnp.zeros((16,), jnp.float32)
    izero16 = jnp.zeros((16,), jnp.int32)

    def _zrow(r, _):
        for j in range(CH // 16):
            zbuf[r, pl.ds(j * 16, 16)] = zero16
        return 0
    lax.fori_loop(0, ZCH, _zrow, 0)
    for i in range(ROWS_PER_TILE // ZCH):
        pltpu.sync_copy(zbuf, aggsh.at[pl.ds(s * ROWS_PER_TILE + i * ZCH, ZCH)])
    for j in range(C // 16):
        zacc[pl.ds(j * 16, 16)] = zero16
    for g in range(NG):
        gs = pl.ds(g * 16, 16)
        row_a[gs] = izero16
        row_b[gs] = izero16
    for e in range(CHUNK):
        for j in range(CH // 16):
            wa[e, pl.ds(j * 16, 16)] = zero16
            wb[e, pl.ds(j * 16, 16)] = zero16
    plsc.subcore_barrier()
    pltpu.async_copy(wa, aggsh.at[row_a], sw_a, add=True)
    pltpu.async_copy(wb, aggsh.at[row_b], sw_b, add=True)

    lanes = lax.iota(jnp.int32, 16)

    kvh = kv_hbm.at[c]

    def issue(cc, qd, kvd, semq, semkv):
        o = cc * CHUNK
        di = dst_sv.at[pl.ds(o, CHUNK)]
        si = src_sv.at[pl.ds(o, CHUNK)]
        pltpu.async_copy(q_hbm.at[di], qd, semq)
        pltpu.async_copy(kvh.at[si], kvd, semkv)

    def wait(qd, kvd, semq, semkv):
        dummy = q_hbm.at[pl.ds(0, CHUNK)]
        pltpu.make_async_copy(dummy, qd, semq).wait()
        pltpu.make_async_copy(dummy, kvd, semkv).wait()

    def compute(cc, qd, kvd, wd, rowd, semw):
        pltpu.make_async_copy(wd, aggsh.at[rowd], semw).wait()
        for g in range(NG):
            evec = g * 16 + lanes
            gs = pl.ds(cc * CHUNK + g * 16, 16)
            rowd[pl.ds(g * 16, 16)] = dst_sv[gs]
            for h in range(HB):
                cols = [jnp.full((16,), h * D + d, jnp.int32)
                        for d in range(D)]
                qvs = [plsc.load_gather(qd, [evec, cols[d] + cb])
                       for d in range(D)]
                kvs = [plsc.load_gather(kvd, [evec, cols[d]])
                       for d in range(D)]
                prods = [qvs[d] * kvs[d] for d in range(D)]
                while len(prods) > 1:
                    prods = [prods[i] + prods[i + 1]
                             for i in range(0, len(prods), 2)]
                wvs = [plsc.load_gather(kvd, [evec, cols[d] + CH])
                       for d in range(D)]
                p = jnp.exp(prods[0] * 0.25)
                hs = pl.ds((c * HB + h) * 16, 16)
                zacc[hs] = zacc[hs] + p
                for d in range(D):
                    plsc.store_scatter(wd, [evec, cols[d]], wvs[d] * p)
        pltpu.async_copy(wd, aggsh.at[rowd], semw, add=True)

    def super_body(t, _):
        base = s * EW + t * (SUPER * CHUNK)
        pltpu.sync_copy(src_hbm.at[pl.ds(base, SUPER * CHUNK)], src_sv)
        pltpu.sync_copy(dst_hbm.at[pl.ds(base, SUPER * CHUNK)], dst_sv)

        issue(0, qa, kva, sq_a, skv_a)

        def pair_body(i, _):
            issue(2 * i + 1, qb, kvb, sq_b, skv_b)
            wait(qa, kva, sq_a, skv_a)
            compute(2 * i, qa, kva, wa, row_a, sw_a)
            issue(2 * i + 2, qa, kva, sq_a, skv_a)
            wait(qb, kvb, sq_b, skv_b)
            compute(2 * i + 1, qb, kvb, wb, row_b, sw_b)
            return 0
        lax.fori_loop(0, SUPER // 2, pair_body, 0)

        wait(qa, kva, sq_a, skv_a)
        compute(SUPER - 1, qa, kva, wa, row_a, sw_a)
        return 0

    lax.fori_loop(0, NSUP, super_body, 0)

    pltpu.make_async_copy(wa, aggsh.at[row_a], sw_a).wait()
    pltpu.make_async_copy(wb, aggsh.at[row_b], sw_b).wait()

    pltpu.sync_copy(zacc, z_out.at[c * NS + s])
    plsc.subcore_barrier()
    pltpu.sync_copy(aggsh.at[pl.ds(s * ROWS_PER_TILE, ROWS_PER_TILE)],
                    agg_out.at[c, pl.ds(s * ROWS_PER_TILE, ROWS_PER_TILE)])


@functools.partial(
    pl.kernel,
    out_type=(jax.ShapeDtypeStruct((2, AGG_ROWS, CH), jnp.float32),
              jax.ShapeDtypeStruct((2 * NS, C), jnp.float32)),
    mesh=plsc.VectorSubcoreMesh(core_axis_name="c", subcore_axis_name="s"),
    scratch_types=(
        [pltpu.VMEM((SUPER * CHUNK,), jnp.int32)] * 2
        + [pltpu.VMEM((CHUNK,), jnp.int32)] * 2
        + [pltpu.VMEM((CHUNK, C), jnp.float32)] * 2
        + [pltpu.VMEM((CHUNK, CH), jnp.float32)]
        + [pltpu.VMEM((CHUNK, C), jnp.float32)] * 2
        + [pltpu.VMEM((CHUNK, CH), jnp.float32)]
        + [pltpu.VMEM((C,), jnp.float32),
           pltpu.VMEM((ZCH, CH), jnp.float32),
           pltpu.VMEM_SHARED((AGG_ROWS, CH), jnp.float32)]
        + [pltpu.SemaphoreType.DMA] * 6
    ),
    compiler_params=pltpu.CompilerParams(needs_layout_passes=False),
)
def _sc_attention(q_hbm, kv_hbm, src_hbm, dst_hbm, agg_out, z_out, *rest):
    _sc_body(q_hbm, kv_hbm, src_hbm, dst_hbm, agg_out, z_out, *rest)


def _epilogue_body(x_ref, a0_ref, a1_ref, scale_ref, wo_ref, bo_ref,
                   g2_ref, b2_ref, w1_ref, b1_ref, w2_ref, bf2_ref, out_ref):
    agg = jnp.concatenate((a0_ref[...], a1_ref[...]), axis=1) * scale_ref[...]
    o = jnp.dot(agg, wo_ref[...], preferred_element_type=jnp.float32)
    x2 = x_ref[...] + o + bo_ref[...]
    mu = jnp.mean(x2, axis=-1, keepdims=True)
    var = jnp.mean((x2 - mu) * (x2 - mu), axis=-1, keepdims=True)
    xn2 = (x2 - mu) * lax.rsqrt(var + 1e-5) * g2_ref[...] + b2_ref[...]
    h = jnp.dot(xn2, w1_ref[...], preferred_element_type=jnp.float32)
    h = jnp.maximum(h + b1_ref[...], 0.0)
    f = jnp.dot(h, w2_ref[...], preferred_element_type=jnp.float32)
    out_ref[...] = x2 + f + bf2_ref[...]


def _epilogue(x, a0, a1, scale, wo, bo, g2, b2, w1, b1, w2, bf2):
    full = lambda r, c: pl.BlockSpec((r, c), lambda i: (0, 0))
    return pl.pallas_call(
        _epilogue_body,
        grid=(N // BLK,),
        in_specs=[
            pl.BlockSpec((BLK, C), lambda i: (i, 0)),
            pl.BlockSpec((BLK, CH), lambda i: (i, 0)),
            pl.BlockSpec((BLK, CH), lambda i: (i, 0)),
            full(1, C),
            full(C, C),
            full(1, C),
            full(1, C),
            full(1, C),
            full(C, 4 * C),
            full(1, 4 * C),
            full(4 * C, C),
            full(1, C),
        ],
        out_specs=pl.BlockSpec((BLK, C), lambda i: (i, 0)),
        out_shape=jax.ShapeDtypeStruct((N, C), jnp.float32),
    )(x, a0, a1, scale, wo, bo, g2, b2, w1, b1, w2, bf2)


def kernel(x, edge_index, Wq, bq, Wk, bk, Wv, bv, Wo, bo, ln1_g, ln1_b,
           ln2_g, ln2_b, W1, b1, W2, b2):
    wqkv = jnp.concatenate([Wq, Wk, Wv], axis=1)
    bqkv = jnp.concatenate([bq, bk, bv]).reshape(1, 3 * C)
    q, kvp = _prologue(x, wqkv, bqkv, ln1_g.reshape(1, C), ln1_b.reshape(1, C))

    src = edge_index[0].astype(jnp.int32)
    dst = edge_index[1].astype(jnp.int32)
    aggp, zp = _sc_attention(q, kvp, src, dst)

    z = zp.sum(axis=0).reshape(H, D).sum(-1)
    scale = jnp.repeat(1.0 / z, D).reshape(1, C)

    return _epilogue(x, aggp[0], aggp[1], scale, Wo, bo.reshape(1, C),
                     ln2_g.reshape(1, C), ln2_b.reshape(1, C),
                     W1, b1.reshape(1, 4 * C), W2, b2.reshape(1, C))
